# baseline (speedup 1.0000x reference)
import functools

import jax
import jax.numpy as jnp
from jax.experimental import pallas as pl
from jax.experimental.pallas import tpu as pltpu

S = 2048
D_MODEL = 768
D_FFN = 1536
N_EXPERTS = 8
TOP_K = 2

TBLK = 256
EPAD = 128


def _router_comb(x, choice_p):
    logits = jax.lax.dot_general(
        x, choice_p, (((1,), (1,)), ((), ())),
        preferred_element_type=jnp.float32,
    )
    eiota = jax.lax.broadcasted_iota(jnp.int32, logits.shape, 1)
    valid = eiota < N_EXPERTS
    logits = jnp.where(valid, logits, -jnp.inf)
    m = jnp.max(logits, axis=1, keepdims=True)
    p = jnp.exp(logits - m)
    probs = p / jnp.sum(p, axis=1, keepdims=True)
    m1 = jnp.max(probs, axis=1, keepdims=True)
    i1 = jnp.min(jnp.where(probs == m1, eiota, EPAD), axis=1, keepdims=True)
    mask1 = eiota == i1
    probs2 = jnp.where(mask1 | ~valid, -1.0, probs)
    m2 = jnp.max(probs2, axis=1, keepdims=True)
    i2 = jnp.min(jnp.where(probs2 == m2, eiota, EPAD), axis=1, keepdims=True)
    return probs * ((mask1 | (eiota == i2)).astype(jnp.float32))


def _moe_body(x_ref, choice_ref, w1_ref, w2_ref, out_ref, acc_ref, comb_ref):
    e = pl.program_id(0)
    t = pl.program_id(1)
    x = x_ref[...]

    @pl.when(e == 0)
    def _():
        comb_ref[pl.ds(t * TBLK, TBLK), :] = _router_comb(x, choice_ref[...])

    h = jax.lax.dot_general(
        x, w1_ref[0], (((1,), (1,)), ((), ())),
        preferred_element_type=jnp.float32,
    )
    h = h * jax.nn.sigmoid(h)
    y = jax.lax.dot_general(
        h, w2_ref[0], (((1,), (1,)), ((), ())),
        preferred_element_type=jnp.float32,
    )

    cvec = comb_ref[pl.ds(t * TBLK, TBLK), :]
    lane = jax.lax.broadcasted_iota(jnp.int32, cvec.shape, 1)
    scale = jnp.sum(jnp.where(lane == e, cvec, 0.0), axis=1, keepdims=True)
    contrib = scale * y

    @pl.when(e == 0)
    def _():
        acc_ref[pl.ds(t * TBLK, TBLK), :] = contrib

    @pl.when(e > 0)
    def _():
        acc_ref[pl.ds(t * TBLK, TBLK), :] += contrib

    @pl.when(e == N_EXPERTS - 1)
    def _():
        out_ref[...] = acc_ref[pl.ds(t * TBLK, TBLK), :]


@jax.jit
def kernel(x, choice, w1, w2):
    b, s, d = x.shape
    x2 = x.reshape(s, d)
    choice_p = jnp.zeros((EPAD, D_MODEL), jnp.float32).at[:N_EXPERTS].set(choice)
    out = pl.pallas_call(
        _moe_body,
        grid=(N_EXPERTS, s // TBLK),
        in_specs=[
            pl.BlockSpec((TBLK, D_MODEL), lambda e, t: (t, 0)),
            pl.BlockSpec((EPAD, D_MODEL), lambda e, t: (0, 0)),
            pl.BlockSpec((1, D_FFN, D_MODEL), lambda e, t: (e, 0, 0)),
            pl.BlockSpec((1, D_MODEL, D_FFN), lambda e, t: (e, 0, 0)),
        ],
        out_specs=pl.BlockSpec((TBLK, D_MODEL), lambda e, t: (t, 0)),
        out_shape=jax.ShapeDtypeStruct((s, d), jnp.float32),
        scratch_shapes=[
            pltpu.VMEM((S, D_MODEL), jnp.float32),
            pltpu.VMEM((S, EPAD), jnp.float32),
        ],
        compiler_params=pltpu.CompilerParams(
            dimension_semantics=("arbitrary", "arbitrary"),
        ),
    )(x2, choice_p, w1, w2)
    return out.reshape(b, s, d)
